# Initial kernel scaffold; baseline (speedup 1.0000x reference)
import jax
import jax.numpy as jnp
from jax.experimental import pallas as pl


def kernel(z, att_w1, att_b1, att_w2, att_b2, mlp_w1, mlp_b1, mlp_w2, mlp_b2, mlp_w3, mlp_b3):
    raise NotImplementedError("write your pallas kernel here")


# speedup vs baseline: 4.9079x; 4.9079x over previous
import jax
import jax.numpy as jnp
from jax.experimental import pallas as pl
from jax.experimental.pallas import tpu as pltpu

N = 100000
LATENT = 32
BLOCK = 5000


def _body(z_ref, w1_ref, b1_ref, w2_ref, b2_ref,
          mw1_ref, mb1_ref, mw2_ref, mb2_ref, mw3_ref, mb3_ref,
          out_ref, m_ref, s_ref, v_ref):
    i = pl.program_id(0)

    @pl.when(i == 0)
    def _():
        m_ref[0, 0] = -1e30
        s_ref[0, 0] = 0.0
        v_ref[...] = jnp.zeros_like(v_ref)

    zb = z_ref[...]
    h = jnp.tanh(zb @ w1_ref[...] + b1_ref[...])
    logit = h @ w2_ref[...] + b2_ref[...]

    m_old = m_ref[0, 0]
    m_new = jnp.maximum(m_old, jnp.max(logit))
    scale = jnp.exp(m_old - m_new)
    e = jnp.exp(logit - m_new)
    s_ref[0, 0] = s_ref[0, 0] * scale + jnp.sum(e)
    v_ref[...] = v_ref[...] * scale + jnp.sum(zb * e, axis=0, keepdims=True)
    m_ref[0, 0] = m_new

    @pl.when(i == pl.num_programs(0) - 1)
    def _():
        g = v_ref[...] / (s_ref[0, 0] + 1e-8)
        x = jnp.maximum(g @ mw1_ref[...] + mb1_ref[...], 0.0)
        x = jnp.maximum(x @ mw2_ref[...] + mb2_ref[...], 0.0)
        y = x @ mw3_ref[...] + mb3_ref[...]
        out_ref[...] = jax.nn.sigmoid(y)


def kernel(z, att_w1, att_b1, att_w2, att_b2,
           mlp_w1, mlp_b1, mlp_w2, mlp_b2, mlp_w3, mlp_b3):
    grid = (N // BLOCK,)
    small = lambda shape: pl.BlockSpec(shape, lambda i: tuple(0 for _ in shape))
    out = pl.pallas_call(
        _body,
        grid=grid,
        in_specs=[
            pl.BlockSpec((BLOCK, LATENT), lambda i: (i, 0)),
            small((LATENT, LATENT)),
            small((1, LATENT)),
            small((LATENT, 1)),
            small((1, 1)),
            small((LATENT, 128)),
            small((1, 128)),
            small((128, 64)),
            small((1, 64)),
            small((64, 1)),
            small((1, 1)),
        ],
        out_specs=pl.BlockSpec((1, 1), lambda i: (0, 0)),
        out_shape=jax.ShapeDtypeStruct((1, 1), jnp.float32),
        scratch_shapes=[
            pltpu.SMEM((1, 1), jnp.float32),
            pltpu.SMEM((1, 1), jnp.float32),
            pltpu.VMEM((1, LATENT), jnp.float32),
        ],
        compiler_params=pltpu.CompilerParams(
            dimension_semantics=("arbitrary",),
        ),
    )(
        z,
        att_w1, att_b1.reshape(1, LATENT),
        att_w2, att_b2.reshape(1, 1),
        mlp_w1, mlp_b1.reshape(1, 128),
        mlp_w2, mlp_b2.reshape(1, 64),
        mlp_w3, mlp_b3.reshape(1, 1),
    )
    return out.reshape(-1)


# speedup vs baseline: 5.4389x; 1.1082x over previous
import jax
import jax.numpy as jnp
from jax.experimental import pallas as pl
from jax.experimental.pallas import tpu as pltpu

N = 100000
LATENT = 32
PACK = 4
NP = N // PACK
BLOCK = 5000


def kernel(z, att_w1, att_b1, att_w2, att_b2,
           mlp_w1, mlp_b1, mlp_w2, mlp_b2, mlp_w3, mlp_b3):
    zp = z.reshape(NP, PACK * LATENT)
    eye = jnp.eye(PACK, dtype=jnp.float32)
    w1bd = jnp.kron(eye, att_w1)
    b1t = jnp.tile(att_b1.reshape(1, LATENT), (1, PACK))
    w2s = jnp.kron(eye, att_w2 @ jnp.ones((1, LATENT), jnp.float32))
    small = lambda shape: pl.BlockSpec(shape, lambda i: tuple(0 for _ in shape))

    def body(z_ref, w1_ref, b1_ref, w2s_ref, b2_ref, mw1_ref, mb1_ref,
             mw2_ref, mb2_ref, mw3_ref, mb3_ref, out_ref, m_ref, ve_ref):
        i = pl.program_id(0)

        @pl.when(i == 0)
        def _():
            m_ref[0, 0] = -1e30
            ve_ref[...] = jnp.zeros_like(ve_ref)

        zb = z_ref[...]
        h = jnp.tanh(zb @ w1_ref[...] + b1_ref[...])
        lb = h @ w2s_ref[...] + b2_ref[...]

        m_old = m_ref[0, 0]
        m_new = jnp.maximum(m_old, jnp.max(lb))
        scale = jnp.exp(m_old - m_new)
        eb = jnp.exp(lb - m_new)
        acc = ve_ref[...]
        ve_ref[0:1, :] = acc[0:1, :] * scale + jnp.sum(zb * eb, axis=0, keepdims=True)
        ve_ref[1:2, :] = acc[1:2, :] * scale + jnp.sum(eb, axis=0, keepdims=True)
        m_ref[0, 0] = m_new

        @pl.when(i == pl.num_programs(0) - 1)
        def _():
            acc2 = ve_ref[...]
            folded = (acc2[:, 0:32] + acc2[:, 32:64]
                      + acc2[:, 64:96] + acc2[:, 96:128])
            s = jnp.sum(folded[1:2, :]) * (1.0 / 32.0)
            g = folded[0:1, :] / (s + 1e-8)
            x = jnp.maximum(g @ mw1_ref[...] + mb1_ref[...], 0.0)
            x = jnp.maximum(x @ mw2_ref[...] + mb2_ref[...], 0.0)
            y = x @ mw3_ref[...] + mb3_ref[...]
            out_ref[...] = jax.nn.sigmoid(y)

    out = pl.pallas_call(
        body,
        grid=(NP // BLOCK,),
        in_specs=[
            pl.BlockSpec((BLOCK, PACK * LATENT), lambda i: (i, 0)),
            small((PACK * LATENT, PACK * LATENT)),
            small((1, PACK * LATENT)),
            small((PACK * LATENT, PACK * LATENT)),
            small((1, 1)),
            small((LATENT, 128)),
            small((1, 128)),
            small((128, 64)),
            small((1, 64)),
            small((64, 1)),
            small((1, 1)),
        ],
        out_specs=pl.BlockSpec((1, 1), lambda i: (0, 0)),
        out_shape=jax.ShapeDtypeStruct((1, 1), jnp.float32),
        scratch_shapes=[
            pltpu.SMEM((1, 1), jnp.float32),
            pltpu.VMEM((2, PACK * LATENT), jnp.float32),
        ],
        compiler_params=pltpu.CompilerParams(
            dimension_semantics=("arbitrary",),
        ),
    )(
        zp,
        w1bd, b1t,
        w2s, att_b2.reshape(1, 1),
        mlp_w1, mlp_b1.reshape(1, 128),
        mlp_w2, mlp_b2.reshape(1, 64),
        mlp_w3, mlp_b3.reshape(1, 1),
    )
    return out.reshape(-1)


# speedup vs baseline: 5.7259x; 1.0528x over previous
import jax
import jax.numpy as jnp
from jax.experimental import pallas as pl
from jax.experimental.pallas import tpu as pltpu

N = 100000
LATENT = 32
PACK = 4
NP = N // PACK
BLOCK = 5000


def kernel(z, att_w1, att_b1, att_w2, att_b2,
           mlp_w1, mlp_b1, mlp_w2, mlp_b2, mlp_w3, mlp_b3):
    zp = z.reshape(NP, PACK * LATENT)
    eye = jnp.eye(PACK, dtype=jnp.float32)
    w1bd = jnp.kron(eye, att_w1)
    b1t = jnp.tile(att_b1.reshape(1, LATENT), (1, PACK))
    w2s = jnp.kron(eye, att_w2 @ jnp.ones((1, LATENT), jnp.float32))
    small = lambda shape: pl.BlockSpec(shape, lambda i: tuple(0 for _ in shape))

    def body(z_ref, w1_ref, b1_ref, w2s_ref, mw1_ref, mb1_ref,
             mw2_ref, mb2_ref, mw3_ref, mb3_ref, out_ref, ve_ref):
        i = pl.program_id(0)

        @pl.when(i == 0)
        def _():
            ve_ref[...] = jnp.zeros_like(ve_ref)

        zb = z_ref[...]
        h = jnp.tanh(zb @ w1_ref[...] + b1_ref[...])
        eb = jnp.exp(h @ w2s_ref[...])
        ve_ref[0:1, :] += jnp.sum(zb * eb, axis=0, keepdims=True)
        ve_ref[1:2, :] += jnp.sum(eb, axis=0, keepdims=True)

        @pl.when(i == pl.num_programs(0) - 1)
        def _():
            acc2 = ve_ref[...]
            vz = (acc2[0:1, 0:32] + acc2[0:1, 32:64]
                  + acc2[0:1, 64:96] + acc2[0:1, 96:128])
            s = jnp.sum(acc2[1:2, :]) * (1.0 / 32.0)
            g = vz / (s + 1e-8)
            x = jnp.maximum(g @ mw1_ref[...] + mb1_ref[...], 0.0)
            x = jnp.maximum(x @ mw2_ref[...] + mb2_ref[...], 0.0)
            y = x @ mw3_ref[...] + mb3_ref[...]
            out_ref[...] = jax.nn.sigmoid(y)

    out = pl.pallas_call(
        body,
        grid=(NP // BLOCK,),
        in_specs=[
            pl.BlockSpec((BLOCK, PACK * LATENT), lambda i: (i, 0)),
            small((PACK * LATENT, PACK * LATENT)),
            small((1, PACK * LATENT)),
            small((PACK * LATENT, PACK * LATENT)),
            small((LATENT, 128)),
            small((1, 128)),
            small((128, 64)),
            small((1, 64)),
            small((64, 1)),
            small((1, 1)),
        ],
        out_specs=pl.BlockSpec((1, 1), lambda i: (0, 0)),
        out_shape=jax.ShapeDtypeStruct((1, 1), jnp.float32),
        scratch_shapes=[
            pltpu.VMEM((2, PACK * LATENT), jnp.float32),
        ],
        compiler_params=pltpu.CompilerParams(
            dimension_semantics=("arbitrary",),
        ),
    )(
        zp,
        w1bd, b1t, w2s,
        mlp_w1, mlp_b1.reshape(1, 128),
        mlp_w2, mlp_b2.reshape(1, 64),
        mlp_w3, mlp_b3.reshape(1, 1),
    )
    return out.reshape(-1)


# speedup vs baseline: 22.7047x; 3.9653x over previous
import jax
import jax.numpy as jnp
from jax.experimental import pallas as pl
from jax.experimental.pallas import tpu as pltpu

N = 100000
LATENT = 32
LB = 12800
GRID = (N + LB - 1) // LB


def kernel(z, att_w1, att_b1, att_w2, att_b2,
           mlp_w1, mlp_b1, mlp_w2, mlp_b2, mlp_w3, mlp_b3):
    zt = z.T
    w1t = att_w1.T
    b1c = att_b1.reshape(LATENT, 1)
    w2r = jnp.tile(att_w2.T, (8, 1))
    mw1t = mlp_w1.T
    mb1c = mlp_b1.reshape(128, 1)
    mw2t = mlp_w2.T
    mb2c = mlp_b2.reshape(64, 1)
    mw3t = mlp_w3.T
    small = lambda shape: pl.BlockSpec(shape, lambda i: tuple(0 for _ in shape))

    def body(z_ref, w1_ref, b1_ref, w2_ref, mw1_ref, mb1_ref,
             mw2_ref, mb2_ref, mw3_ref, mb3_ref, out_ref, av_ref, ae_ref):
        i = pl.program_id(0)
        nsteps = pl.num_programs(0)

        @pl.when(i == 0)
        def _():
            av_ref[...] = jnp.zeros_like(av_ref)
            ae_ref[...] = jnp.zeros_like(ae_ref)

        zb = z_ref[...]
        h = jnp.tanh(w1_ref[...] @ zb + b1_ref[...])
        e8 = jnp.exp(w2_ref[...] @ h)

        def accumulate(p, e8v):
            av = av_ref[...]
            ae = ae_ref[...]
            for c in range(LB // 128):
                av += p[:, 128 * c:128 * (c + 1)]
                ae += e8v[:, 128 * c:128 * (c + 1)]
            av_ref[...] = av
            ae_ref[...] = ae

        @pl.when(i < nsteps - 1)
        def _():
            accumulate(zb * e8[0:1, :], e8)

        @pl.when(i == nsteps - 1)
        def _():
            valid = N - (nsteps - 1) * LB
            lane8 = jax.lax.broadcasted_iota(jnp.int32, (8, LB), 1)
            lane32 = jax.lax.broadcasted_iota(jnp.int32, (LATENT, LB), 1)
            e8m = jnp.where(lane8 < valid, e8, 0.0)
            pm = jnp.where(lane32 < valid, zb * e8[0:1, :], 0.0)
            accumulate(pm, e8m)

            s = jnp.sum(ae_ref[...]) * 0.125
            vz = jnp.sum(av_ref[...], axis=1, keepdims=True)
            g = vz / (s + 1e-8)
            x = jnp.maximum(mw1_ref[...] @ g + mb1_ref[...], 0.0)
            x = jnp.maximum(mw2_ref[...] @ x + mb2_ref[...], 0.0)
            y = mw3_ref[...] @ x + mb3_ref[...]
            out_ref[...] = jax.nn.sigmoid(y)

    out = pl.pallas_call(
        body,
        grid=(GRID,),
        in_specs=[
            pl.BlockSpec((LATENT, LB), lambda i: (0, i)),
            small((LATENT, LATENT)),
            small((LATENT, 1)),
            small((8, LATENT)),
            small((128, LATENT)),
            small((128, 1)),
            small((64, 128)),
            small((64, 1)),
            small((1, 64)),
            small((1, 1)),
        ],
        out_specs=pl.BlockSpec((1, 1), lambda i: (0, 0)),
        out_shape=jax.ShapeDtypeStruct((1, 1), jnp.float32),
        scratch_shapes=[
            pltpu.VMEM((LATENT, 128), jnp.float32),
            pltpu.VMEM((8, 128), jnp.float32),
        ],
        compiler_params=pltpu.CompilerParams(
            dimension_semantics=("arbitrary",),
        ),
    )(
        zt, w1t, b1c, w2r,
        mw1t, mb1c, mw2t, mb2c, mw3t, mlp_b3.reshape(1, 1),
    )
    return out.reshape(-1)
